# baseline (speedup 1.0000x reference)
import functools

import jax
import jax.numpy as jnp
from jax import lax
from jax.experimental import pallas as pl
from jax.experimental.pallas import tpu as pltpu
from jax.experimental.pallas import tpu_sc as plsc

_V = 1000000
_B = 425984
_D = 64
_NC = 2
_NS = 16
_NW = _NC * _NS
_CH = 128
_BPW = _B // _NW
_NCHUNK = _BPW // _CH
_NBUF = 4
_NT = 2

_RW = 32768
_RGRID = (_V + _RW - 1) // _RW

_mesh = plsc.VectorSubcoreMesh(core_axis_name="c", subcore_axis_name="s")


def _relayout_body(t_ref, o_ref):
    o_ref[:, 0:_D] = t_ref[...].T


_relayout = pl.pallas_call(
    _relayout_body,
    grid=(_RGRID,),
    in_specs=[pl.BlockSpec((_D, _RW), lambda k: (0, k))],
    out_specs=pl.BlockSpec((_RW, 128), lambda k: (k, 0)),
    out_shape=jax.ShapeDtypeStruct((_V, 128), jnp.float32),
)


_NSPLIT = 2
_BH = _B // _NSPLIT
_BPWH = _BH // _NW
_NCHUNKH = _BPWH // _CH


def _make_gather(base0):
    @functools.partial(
        pl.kernel,
        out_type=jax.ShapeDtypeStruct((_BH, _D), jnp.float32),
        mesh=_mesh,
        compiler_params=pltpu.CompilerParams(needs_layout_passes=False),
        scratch_types=[
            pltpu.VMEM((_BPWH,), jnp.int32),
            pltpu.VMEM((_NBUF, _CH, 128), jnp.float32),
            pltpu.VMEM((_NT, _CH, _D), jnp.float32),
            pltpu.SemaphoreType.DMA,
            pltpu.SemaphoreType.DMA,
        ],
    )
    def _gather_kernel(idx_hbm, tablep_hbm, out_hbm, idx_v, rows_v, t_v,
                       gsem, osem):
        wid = lax.axis_index("s") * _NC + lax.axis_index("c")
        base = wid * _BPWH
        pltpu.sync_copy(idx_hbm.at[pl.ds(base0 + base, _BPWH)], idx_v)

        for b in range(_NBUF):
            pltpu.async_copy(
                tablep_hbm.at[idx_v.at[pl.ds(b * _CH, _CH)]], rows_v.at[b],
                gsem,
            )

        def group(g, carry):
            for b in range(_NBUF):
                j = g * _NBUF + b
                tb = b % _NT
                pltpu.make_async_copy(
                    tablep_hbm.at[idx_v.at[pl.ds(0, _CH)]], rows_v.at[b], gsem
                ).wait()

                @pl.when(j >= _NT)
                def _():
                    pltpu.make_async_copy(
                        t_v.at[0], out_hbm.at[pl.ds(base, _CH)], osem
                    ).wait()

                def pbody(j8, c):
                    for j_lo in range(8):
                        for dg in range(_D // 16):
                            t_v[tb, j8 * 8 + j_lo, pl.ds(dg * 16, 16)] = (
                                rows_v[b, j8 * 8 + j_lo, pl.ds(dg * 16, 16)]
                            )
                    return c

                lax.fori_loop(0, _CH // 8, pbody, 0)

                pltpu.async_copy(
                    t_v.at[tb], out_hbm.at[pl.ds(base + j * _CH, _CH)], osem
                )
                nxt = j + _NBUF

                @pl.when(nxt < _NCHUNKH)
                def _():
                    pltpu.async_copy(
                        tablep_hbm.at[idx_v.at[pl.ds(nxt * _CH, _CH)]],
                        rows_v.at[b], gsem,
                    )

            return carry

        lax.fori_loop(0, _NCHUNKH // _NBUF, group, 0)

        for tb in range(_NT):
            pltpu.make_async_copy(
                t_v.at[tb], out_hbm.at[pl.ds(base, _CH)], osem
            ).wait()

    return _gather_kernel


_gather_halves = [_make_gather(h * _BH) for h in range(_NSPLIT)]


def kernel(indice, embedding_table):
    idx = indice.astype(jnp.int32)
    tablep = _relayout(embedding_table.T)
    parts = [g(idx, tablep) for g in _gather_halves]
    return jnp.concatenate(parts, axis=0)


# speedup vs baseline: 1.1572x; 1.1572x over previous
import functools

import jax
import jax.numpy as jnp
from jax import lax
from jax.experimental import pallas as pl
from jax.experimental.pallas import tpu as pltpu
from jax.experimental.pallas import tpu_sc as plsc

_V = 1000000
_B = 425984
_D = 64
_NC = 2
_NS = 16
_NW = _NC * _NS
_CH = 128
_BPW = _B // _NW
_NCHUNK = _BPW // _CH
_NBUF = 4
_NT = 2

_RW = 32768
_RGRID = (_V + _RW - 1) // _RW

_mesh = plsc.VectorSubcoreMesh(core_axis_name="c", subcore_axis_name="s")


def _relayout_body(t_ref, o_ref):
    o_ref[:, 0:_D] = t_ref[...].T


_relayout = pl.pallas_call(
    _relayout_body,
    grid=(_RGRID,),
    in_specs=[pl.BlockSpec((_D, _RW), lambda k: (0, k))],
    out_specs=pl.BlockSpec((_RW, 128), lambda k: (k, 0)),
    out_shape=jax.ShapeDtypeStruct((_V, 128), jnp.float32),
)


@functools.partial(
    pl.kernel,
    out_type=jax.ShapeDtypeStruct((_B, _D), jnp.float32),
    mesh=_mesh,
    compiler_params=pltpu.CompilerParams(needs_layout_passes=False),
    scratch_types=[
        pltpu.VMEM((_BPW,), jnp.int32),
        pltpu.VMEM((_NBUF, _CH, 128), jnp.float32),
        pltpu.VMEM((_NT, _CH, _D), jnp.float32),
        pltpu.SemaphoreType.DMA,
        pltpu.SemaphoreType.DMA,
    ],
)
def _gather_kernel(idx_hbm, tablep_hbm, out_hbm, idx_v, rows_v, t_v, gsem, osem):
    wid = lax.axis_index("s") * _NC + lax.axis_index("c")
    base = wid * _BPW
    pltpu.sync_copy(idx_hbm.at[pl.ds(base, _BPW)], idx_v)

    for b in range(_NBUF):
        pltpu.async_copy(
            tablep_hbm.at[idx_v.at[pl.ds(b * _CH, _CH)]], rows_v.at[b], gsem
        )

    def group(g, carry):
        for b in range(_NBUF):
            j = g * _NBUF + b
            tb = b % _NT
            pltpu.make_async_copy(
                tablep_hbm.at[idx_v.at[pl.ds(0, _CH)]], rows_v.at[b], gsem
            ).wait()

            @pl.when(j >= _NT)
            def _():
                pltpu.make_async_copy(
                    t_v.at[0], out_hbm.at[pl.ds(base, _CH)], osem
                ).wait()

            def pbody(j8, c):
                for j_lo in range(8):
                    for dg in range(_D // 16):
                        t_v[tb, j8 * 8 + j_lo, pl.ds(dg * 16, 16)] = rows_v[
                            b, j8 * 8 + j_lo, pl.ds(dg * 16, 16)
                        ]
                return c

            lax.fori_loop(0, _CH // 8, pbody, 0)

            pltpu.async_copy(
                t_v.at[tb], out_hbm.at[pl.ds(base + j * _CH, _CH)], osem
            )
            nxt = j + _NBUF

            @pl.when(nxt < _NCHUNK)
            def _():
                pltpu.async_copy(
                    tablep_hbm.at[idx_v.at[pl.ds(nxt * _CH, _CH)]],
                    rows_v.at[b], gsem,
                )

        return carry

    lax.fori_loop(0, _NCHUNK // _NBUF, group, 0)

    for tb in range(_NT):
        pltpu.make_async_copy(
            t_v.at[tb], out_hbm.at[pl.ds(base, _CH)], osem
        ).wait()


def kernel(indice, embedding_table):
    idx = indice.astype(jnp.int32)
    tablep = _relayout(embedding_table.T)
    return _gather_kernel(idx, tablep)
